# baseline (device time: 77751 ns/iter reference)
import jax
import jax.numpy as jnp
from jax import lax
from jax.experimental import pallas as pl
from jax.experimental.pallas import tpu as pltpu

N_DEV = 16
SQ = 1024
SKV = 1024
DH = 128
H_LOC = 8
D_LOC = H_LOC * DH
N_BAND = 4
BAND = SQ // N_BAND
STRIP = BAND // N_DEV
SCALE = 0.08838834764831843
BLK = 64


def _attn_band(k, q_band, k_all, v_all, mask_q):
    lo = k * BAND
    ext = lo + BAND
    s = lax.dot_general(q_band, k_all[:, :ext, :], (((2,), (2,)), ((0,), (0,))),
                        preferred_element_type=jnp.float32) * SCALE
    w_r = jnp.where(mask_q[None], jnp.exp(s[:, :, lo:]), 0.0)
    denom = jnp.sum(w_r, axis=2, keepdims=True)
    ctx = lax.dot_general(w_r.astype(jnp.bfloat16), v_all[:, lo:ext, :],
                          (((2,), (1,)), ((0,), (0,))),
                          preferred_element_type=jnp.float32)
    if k > 0:
        w_l = jnp.exp(s[:, :, :lo])
        denom = denom + jnp.sum(w_l, axis=2, keepdims=True)
        ctx = ctx + lax.dot_general(w_l.astype(jnp.bfloat16), v_all[:, :lo, :],
                                    (((2,), (1,)), ((0,), (0,))),
                                    preferred_element_type=jnp.float32)
    return ctx / denom


def _body(x_ref, wq_ref, k_hbm, v_hbm, wo_ref, out_ref,
          kf_ref, vf_ref, kb_ref, vb_ref, wob_ref, stage_ref, red_ref,
          p1_buf,
          k_sems, v_sems, p1_send, p1_recv, p2_send, p2_recv):
    d = lax.axis_index("i")

    def strip_rows(band, dev):
        return pl.ds(band * BAND + dev * STRIP, STRIP)

    kv_copies = []
    for h in range(H_LOC):
        idx = d * H_LOC + h
        ck = pltpu.make_async_copy(k_hbm.at[0, :, idx, :], kf_ref.at[h],
                                   k_sems.at[h])
        cv = pltpu.make_async_copy(v_hbm.at[0, :, idx, :], vf_ref.at[h],
                                   v_sems.at[h])
        ck.start()
        cv.start()
        kv_copies += [ck, cv]

    q = jnp.dot(x_ref[0].astype(jnp.bfloat16), wq_ref[...].astype(jnp.bfloat16),
                preferred_element_type=jnp.float32)
    q_all = q.astype(jnp.bfloat16).reshape(SQ, H_LOC, DH).transpose(1, 0, 2)
    wob_ref[...] = wo_ref[...].astype(jnp.bfloat16)

    ri = lax.broadcasted_iota(jnp.int32, (BAND, BAND), 0)
    ci = lax.broadcasted_iota(jnp.int32, (BAND, BAND), 1)
    mask_q = (ri // BLK) >= (ci // BLK)

    for cp in kv_copies:
        cp.wait()
    kb_ref[...] = kf_ref[...].astype(jnp.bfloat16)
    vb_ref[...] = vf_ref[...].astype(jnp.bfloat16)
    k_all = kb_ref[...]
    v_all = vb_ref[...]

    p1 = {}
    for band in range(N_BAND):
        ctx_k = _attn_band(band, q_all[:, band * BAND:(band + 1) * BAND, :],
                           k_all, v_all, mask_q)
        ctx_k = ctx_k.astype(jnp.bfloat16).transpose(1, 0, 2).reshape(BAND,
                                                                      D_LOC)
        stage_ref[band] = jnp.dot(ctx_k, wob_ref[...],
                                  preferred_element_type=jnp.float32
                                  ).astype(jnp.bfloat16)
        rdmas = []
        for off in range(1, N_DEV):
            p = lax.rem(d + off, N_DEV)
            rdma = pltpu.make_async_remote_copy(
                src_ref=stage_ref.at[band, pl.ds(p * STRIP, STRIP), :],
                dst_ref=p1_buf.at[band, off - 1],
                send_sem=p1_send.at[band, off - 1],
                recv_sem=p1_recv.at[band, off - 1],
                device_id=(p,),
                device_id_type=pl.DeviceIdType.MESH,
            )
            rdma.start()
            rdmas.append(rdma)
        p1[band] = rdmas

    p2 = []
    for band in range(N_BAND):
        red = stage_ref[band, pl.ds(d * STRIP, STRIP), :].astype(jnp.float32)
        for off in range(1, N_DEV):
            p1[band][off - 1].wait_recv()
            red = red + p1_buf[band, off - 1].astype(jnp.float32)
        red_ref[band] = red.astype(jnp.bfloat16)
        out_ref[strip_rows(band, d), :] = red_ref[band]
        for off in range(1, N_DEV):
            p = lax.rem(d + off, N_DEV)
            rdma = pltpu.make_async_remote_copy(
                src_ref=red_ref.at[band],
                dst_ref=out_ref.at[strip_rows(band, d), :],
                send_sem=p2_send.at[band, off - 1],
                recv_sem=p2_recv.at[band, off - 1],
                device_id=(p,),
                device_id_type=pl.DeviceIdType.MESH,
            )
            rdma.start()
            p2.append(rdma)

    for band in range(N_BAND):
        for off in range(1, N_DEV):
            src = lax.rem(d + N_DEV - off, N_DEV)
            rdma = pltpu.make_async_remote_copy(
                src_ref=red_ref.at[band],
                dst_ref=out_ref.at[strip_rows(band, src), :],
                send_sem=p1_send.at[band, off - 1],
                recv_sem=p2_recv.at[band, off - 1],
                device_id=(src,),
                device_id_type=pl.DeviceIdType.MESH,
            )
            rdma.wait_recv()

    for band in range(N_BAND):
        for rdma in p1[band]:
            rdma.wait_send()
    for rdma in p2:
        rdma.wait_send()


def kernel(x, Wq, K_ext, V_ext, Wo):
    out = pl.pallas_call(
        _body,
        out_shape=jax.ShapeDtypeStruct((SQ, SQ), jnp.bfloat16),
        in_specs=[
            pl.BlockSpec(memory_space=pltpu.VMEM),
            pl.BlockSpec(memory_space=pltpu.VMEM),
            pl.BlockSpec(memory_space=pl.ANY),
            pl.BlockSpec(memory_space=pl.ANY),
            pl.BlockSpec(memory_space=pltpu.VMEM),
        ],
        out_specs=pl.BlockSpec(memory_space=pltpu.VMEM),
        scratch_shapes=[
            pltpu.VMEM((H_LOC, SKV, DH), jnp.float32),
            pltpu.VMEM((H_LOC, SKV, DH), jnp.float32),
            pltpu.VMEM((H_LOC, SKV, DH), jnp.bfloat16),
            pltpu.VMEM((H_LOC, SKV, DH), jnp.bfloat16),
            pltpu.VMEM((SQ, SQ), jnp.bfloat16),
            pltpu.VMEM((N_BAND, BAND, SQ), jnp.bfloat16),
            pltpu.VMEM((N_BAND, STRIP, SQ), jnp.bfloat16),
            pltpu.VMEM((N_BAND, N_DEV - 1, STRIP, SQ), jnp.bfloat16),
            pltpu.SemaphoreType.DMA((H_LOC,)),
            pltpu.SemaphoreType.DMA((H_LOC,)),
            pltpu.SemaphoreType.DMA((N_BAND, N_DEV - 1)),
            pltpu.SemaphoreType.DMA((N_BAND, N_DEV - 1)),
            pltpu.SemaphoreType.DMA((N_BAND, N_DEV - 1)),
            pltpu.SemaphoreType.DMA((N_BAND, N_DEV - 1)),
        ],
        compiler_params=pltpu.CompilerParams(
            vmem_limit_bytes=110 * 1024 * 1024,
        ),
    )(x, Wq, K_ext, V_ext, Wo)
    return out[None, :, :]


# device time: 70105 ns/iter; 1.1091x vs baseline; 1.1091x over previous
import jax
import jax.numpy as jnp
from jax import lax
from jax.experimental import pallas as pl
from jax.experimental.pallas import tpu as pltpu

N_DEV = 16
N_Z = 4
N_P = 4
SQ = 1024
SKV = 1024
DH = 128
H_LOC = 8
D_LOC = H_LOC * DH
N_BAND = 4
BAND = SQ // N_BAND
CHUNK = SQ // N_DEV
SCALE = 0.08838834764831843
BLK = 64


def _attn_band(k, q_band, k_all, v_all, mask_q):
    lo = k * BAND
    ext = lo + BAND
    s = lax.dot_general(q_band, k_all[:, :ext, :], (((2,), (2,)), ((0,), (0,))),
                        preferred_element_type=jnp.float32) * SCALE
    w_r = jnp.where(mask_q[None], jnp.exp(s[:, :, lo:]), 0.0)
    denom = jnp.sum(w_r, axis=2, keepdims=True)
    ctx = lax.dot_general(w_r.astype(jnp.bfloat16), v_all[:, lo:ext, :],
                          (((2,), (1,)), ((0,), (0,))),
                          preferred_element_type=jnp.float32)
    if k > 0:
        w_l = jnp.exp(s[:, :, :lo])
        denom = denom + jnp.sum(w_l, axis=2, keepdims=True)
        ctx = ctx + lax.dot_general(w_l.astype(jnp.bfloat16), v_all[:, :lo, :],
                                    (((2,), (1,)), ((0,), (0,))),
                                    preferred_element_type=jnp.float32)
    return ctx / denom


def _body(x_ref, wq_ref, k_hbm, v_hbm, wo_ref, out_ref,
          kf_ref, vf_ref, kb_ref, vb_ref, wob_ref, stage_ref,
          buf1, cb_ref, buf2, red_ref, all4_ref, buf3,
          k_sems, v_sems, s1_send, s1_recv, s2_send, s2_recv,
          s3_send, s3_recv, s4_send, s4_recv):
    d = lax.axis_index("i")
    z = lax.div(d, N_P)
    p = lax.rem(d, N_P)

    kv_copies = []
    for h in range(H_LOC):
        idx = d * H_LOC + h
        ck = pltpu.make_async_copy(k_hbm.at[0, :, idx, :], kf_ref.at[h],
                                   k_sems.at[h])
        cv = pltpu.make_async_copy(v_hbm.at[0, :, idx, :], vf_ref.at[h],
                                   v_sems.at[h])
        ck.start()
        cv.start()
        kv_copies += [ck, cv]

    q = jnp.dot(x_ref[0].astype(jnp.bfloat16), wq_ref[...].astype(jnp.bfloat16),
                preferred_element_type=jnp.float32)
    q_all = q.astype(jnp.bfloat16).reshape(SQ, H_LOC, DH).transpose(1, 0, 2)
    wob_ref[...] = wo_ref[...].astype(jnp.bfloat16)

    ri = lax.broadcasted_iota(jnp.int32, (BAND, BAND), 0)
    ci = lax.broadcasted_iota(jnp.int32, (BAND, BAND), 1)
    mask_q = (ri // BLK) >= (ci // BLK)

    for cp in kv_copies:
        cp.wait()
    kb_ref[...] = kf_ref[...].astype(jnp.bfloat16)
    vb_ref[...] = vf_ref[...].astype(jnp.bfloat16)
    k_all = kb_ref[...]
    v_all = vb_ref[...]

    s1 = []
    for band in range(N_BAND):
        ctx_k = _attn_band(band, q_all[:, band * BAND:(band + 1) * BAND, :],
                           k_all, v_all, mask_q)
        ctx_k = ctx_k.astype(jnp.bfloat16).transpose(1, 0, 2).reshape(BAND,
                                                                      D_LOC)
        stage_ref[pl.ds(band * BAND, BAND), :] = jnp.dot(
            ctx_k, wob_ref[...],
            preferred_element_type=jnp.float32).astype(jnp.bfloat16)
        for poff in range(1, N_P):
            pp = lax.rem(p + poff, N_P)
            rdma = pltpu.make_async_remote_copy(
                src_ref=stage_ref.at[pl.ds(band * BAND + pp * CHUNK, CHUNK), :],
                dst_ref=buf1.at[poff - 1, pl.ds(band * CHUNK, CHUNK), :],
                send_sem=s1_send.at[poff - 1, band],
                recv_sem=s1_recv.at[poff - 1, band],
                device_id=(4 * z + pp,),
                device_id_type=pl.DeviceIdType.MESH,
            )
            rdma.start()
            s1.append(rdma)

    for rdma in s1:
        rdma.wait_recv()
    mine = jnp.stack([stage_ref[pl.ds(zc * BAND + p * CHUNK, CHUNK), :]
                      for zc in range(N_Z)]).reshape(N_Z * CHUNK, SQ)
    cb = mine.astype(jnp.float32)
    for poff in range(1, N_P):
        cb = cb + buf1[poff - 1].astype(jnp.float32)
    cb_ref[...] = cb.astype(jnp.bfloat16)

    s2 = []
    for zoff in range(1, N_Z):
        zc = lax.rem(z + zoff, N_Z)
        rdma = pltpu.make_async_remote_copy(
            src_ref=cb_ref.at[pl.ds(zc * CHUNK, CHUNK), :],
            dst_ref=buf2.at[zoff - 1],
            send_sem=s2_send.at[zoff - 1],
            recv_sem=s2_recv.at[zoff - 1],
            device_id=(4 * zc + p,),
            device_id_type=pl.DeviceIdType.MESH,
        )
        rdma.start()
        s2.append(rdma)

    red = cb_ref[pl.ds(z * CHUNK, CHUNK), :].astype(jnp.float32)
    for zoff in range(1, N_Z):
        s2[zoff - 1].wait_recv()
        red = red + buf2[zoff - 1].astype(jnp.float32)
    red_ref[...] = red.astype(jnp.bfloat16)
    all4_ref[pl.ds(z * CHUNK, CHUNK), :] = red_ref[...]

    s3 = []
    for zoff in range(1, N_Z):
        zc = lax.rem(z + zoff, N_Z)
        rdma = pltpu.make_async_remote_copy(
            src_ref=red_ref,
            dst_ref=all4_ref.at[pl.ds(z * CHUNK, CHUNK), :],
            send_sem=s3_send.at[zoff - 1],
            recv_sem=s3_recv.at[zoff - 1],
            device_id=(4 * zc + p,),
            device_id_type=pl.DeviceIdType.MESH,
        )
        rdma.start()
        s3.append(rdma)
    for zoff in range(1, N_Z):
        zs = lax.rem(z + N_Z - zoff, N_Z)
        rdma = pltpu.make_async_remote_copy(
            src_ref=red_ref,
            dst_ref=all4_ref.at[pl.ds(zs * CHUNK, CHUNK), :],
            send_sem=s2_send.at[zoff - 1],
            recv_sem=s3_recv.at[zoff - 1],
            device_id=(4 * zs + p,),
            device_id_type=pl.DeviceIdType.MESH,
        )
        rdma.wait_recv()

    s4 = []
    for poff in range(1, N_P):
        pp = lax.rem(p + poff, N_P)
        rdma = pltpu.make_async_remote_copy(
            src_ref=all4_ref,
            dst_ref=buf3.at[poff - 1],
            send_sem=s4_send.at[poff - 1],
            recv_sem=s4_recv.at[poff - 1],
            device_id=(4 * z + pp,),
            device_id_type=pl.DeviceIdType.MESH,
        )
        rdma.start()
        s4.append(rdma)

    for zc in range(N_Z):
        out_ref[pl.ds(zc * BAND + p * CHUNK, CHUNK), :] = (
            all4_ref[pl.ds(zc * CHUNK, CHUNK), :])
    for poff in range(1, N_P):
        rdma = pltpu.make_async_remote_copy(
            src_ref=all4_ref,
            dst_ref=buf3.at[poff - 1],
            send_sem=s2_send.at[poff - 1],
            recv_sem=s4_recv.at[poff - 1],
            device_id=(d,),
            device_id_type=pl.DeviceIdType.MESH,
        )
        rdma.wait_recv()
        ps = lax.rem(p + N_P - poff, N_P)
        for zc in range(N_Z):
            out_ref[pl.ds(zc * BAND + ps * CHUNK, CHUNK), :] = (
                buf3[poff - 1, pl.ds(zc * CHUNK, CHUNK), :])

    for rdma in s1 + s2 + s3 + s4:
        rdma.wait_send()


def kernel(x, Wq, K_ext, V_ext, Wo):
    out = pl.pallas_call(
        _body,
        out_shape=jax.ShapeDtypeStruct((SQ, SQ), jnp.bfloat16),
        in_specs=[
            pl.BlockSpec(memory_space=pltpu.VMEM),
            pl.BlockSpec(memory_space=pltpu.VMEM),
            pl.BlockSpec(memory_space=pl.ANY),
            pl.BlockSpec(memory_space=pl.ANY),
            pl.BlockSpec(memory_space=pltpu.VMEM),
        ],
        out_specs=pl.BlockSpec(memory_space=pltpu.VMEM),
        scratch_shapes=[
            pltpu.VMEM((H_LOC, SKV, DH), jnp.float32),
            pltpu.VMEM((H_LOC, SKV, DH), jnp.float32),
            pltpu.VMEM((H_LOC, SKV, DH), jnp.bfloat16),
            pltpu.VMEM((H_LOC, SKV, DH), jnp.bfloat16),
            pltpu.VMEM((SQ, SQ), jnp.bfloat16),
            pltpu.VMEM((SQ, SQ), jnp.bfloat16),
            pltpu.VMEM((N_P - 1, N_Z * CHUNK, SQ), jnp.bfloat16),
            pltpu.VMEM((N_Z * CHUNK, SQ), jnp.bfloat16),
            pltpu.VMEM((N_Z - 1, CHUNK, SQ), jnp.bfloat16),
            pltpu.VMEM((CHUNK, SQ), jnp.bfloat16),
            pltpu.VMEM((N_Z * CHUNK, SQ), jnp.bfloat16),
            pltpu.VMEM((N_P - 1, N_Z * CHUNK, SQ), jnp.bfloat16),
            pltpu.SemaphoreType.DMA((H_LOC,)),
            pltpu.SemaphoreType.DMA((H_LOC,)),
            pltpu.SemaphoreType.DMA((N_P - 1, N_BAND)),
            pltpu.SemaphoreType.DMA((N_P - 1, N_BAND)),
            pltpu.SemaphoreType.DMA((N_Z - 1,)),
            pltpu.SemaphoreType.DMA((N_Z - 1,)),
            pltpu.SemaphoreType.DMA((N_Z - 1,)),
            pltpu.SemaphoreType.DMA((N_Z - 1,)),
            pltpu.SemaphoreType.DMA((N_P - 1,)),
            pltpu.SemaphoreType.DMA((N_P - 1,)),
        ],
        compiler_params=pltpu.CompilerParams(
            vmem_limit_bytes=110 * 1024 * 1024,
        ),
    )(x, Wq, K_ext, V_ext, Wo)
    return out[None, :, :]


# device time: 65877 ns/iter; 1.1802x vs baseline; 1.0642x over previous
import jax
import jax.numpy as jnp
from jax import lax
from jax.experimental import pallas as pl
from jax.experimental.pallas import tpu as pltpu

N_DEV = 16
N_Z = 4
N_P = 4
SQ = 1024
SKV = 1024
DH = 128
H_LOC = 8
D_LOC = H_LOC * DH
N_BAND = 4
BAND = SQ // N_BAND
CHUNK = SQ // N_DEV
SCALE = 0.08838834764831843
BLK = 64


def _attn_band(k, q_band, k_all, v_all, mask_q):
    lo = k * BAND
    ext = lo + BAND
    s = lax.dot_general(q_band, k_all[:, :ext, :], (((2,), (2,)), ((0,), (0,))),
                        preferred_element_type=jnp.float32) * SCALE
    w_r = jnp.where(mask_q[None], jnp.exp(s[:, :, lo:]), 0.0)
    denom = jnp.sum(w_r, axis=2, keepdims=True)
    ctx = lax.dot_general(w_r.astype(jnp.bfloat16), v_all[:, lo:ext, :],
                          (((2,), (1,)), ((0,), (0,))),
                          preferred_element_type=jnp.float32)
    if k > 0:
        w_l = jnp.exp(s[:, :, :lo])
        denom = denom + jnp.sum(w_l, axis=2, keepdims=True)
        ctx = ctx + lax.dot_general(w_l.astype(jnp.bfloat16), v_all[:, :lo, :],
                                    (((2,), (1,)), ((0,), (0,))),
                                    preferred_element_type=jnp.float32)
    return ctx / denom


def _body(x_ref, wq_ref, k_hbm, v_hbm, wo_ref, out_ref,
          kf_ref, vf_ref, kb_ref, vb_ref, wob_ref, stage_ref,
          buf1, cb_ref, buf2, red_ref, all4_ref, buf3,
          k_sems, v_sems, s1_send, s1_recv, s2_send, s2_recv,
          s3_send, s3_recv, s4_send, s4_recv):
    d = lax.axis_index("i")
    z = lax.div(d, N_P)
    p = lax.rem(d, N_P)

    kv_copies = []
    for h in range(H_LOC):
        idx = d * H_LOC + h
        ck = pltpu.make_async_copy(k_hbm.at[0, :, idx, :], kf_ref.at[h],
                                   k_sems.at[h])
        cv = pltpu.make_async_copy(v_hbm.at[0, :, idx, :], vf_ref.at[h],
                                   v_sems.at[h])
        ck.start()
        cv.start()
        kv_copies += [ck, cv]

    q = jnp.dot(x_ref[0].astype(jnp.bfloat16), wq_ref[...].astype(jnp.bfloat16),
                preferred_element_type=jnp.float32)
    q_all = q.astype(jnp.bfloat16).reshape(SQ, H_LOC, DH).transpose(1, 0, 2)
    wob_ref[...] = wo_ref[...].astype(jnp.bfloat16)

    ri = lax.broadcasted_iota(jnp.int32, (BAND, BAND), 0)
    ci = lax.broadcasted_iota(jnp.int32, (BAND, BAND), 1)
    mask_q = (ri // BLK) >= (ci // BLK)

    for cp in kv_copies:
        cp.wait()
    kb_ref[...] = kf_ref[...].astype(jnp.bfloat16)
    vb_ref[...] = vf_ref[...].astype(jnp.bfloat16)
    k_all = kb_ref[...]
    v_all = vb_ref[...]

    s1 = []
    for band in range(N_BAND):
        ctx_k = _attn_band(band, q_all[:, band * BAND:(band + 1) * BAND, :],
                           k_all, v_all, mask_q)
        ctx_k = ctx_k.astype(jnp.bfloat16).transpose(1, 0, 2).reshape(BAND,
                                                                      D_LOC)
        stage_ref[pl.ds(band * BAND, BAND), :] = jnp.dot(
            ctx_k, wob_ref[...],
            preferred_element_type=jnp.float32).astype(jnp.bfloat16)
        for poff in range(1, N_P):
            pp = lax.rem(p + poff, N_P)
            rdma = pltpu.make_async_remote_copy(
                src_ref=stage_ref.at[pl.ds(band * BAND + pp * CHUNK, CHUNK), :],
                dst_ref=buf1.at[poff - 1, pl.ds(band * CHUNK, CHUNK), :],
                send_sem=s1_send.at[poff - 1, band],
                recv_sem=s1_recv.at[poff - 1, band],
                device_id=(4 * z + pp,),
                device_id_type=pl.DeviceIdType.MESH,
            )
            rdma.start()
            s1.append(rdma)

    for rdma in s1:
        rdma.wait_recv()
    mine = jnp.stack([stage_ref[pl.ds(zc * BAND + p * CHUNK, CHUNK), :]
                      for zc in range(N_Z)]).reshape(N_Z * CHUNK, SQ)
    cb = mine.astype(jnp.float32)
    for poff in range(1, N_P):
        cb = cb + buf1[poff - 1].astype(jnp.float32)
    cb_ref[...] = cb.astype(jnp.bfloat16)

    s2 = []
    for zoff in range(1, N_Z):
        zc = lax.rem(z + zoff, N_Z)
        rdma = pltpu.make_async_remote_copy(
            src_ref=cb_ref.at[pl.ds(zc * CHUNK, CHUNK), :],
            dst_ref=buf2.at[zoff - 1],
            send_sem=s2_send.at[zoff - 1],
            recv_sem=s2_recv.at[zoff - 1],
            device_id=(4 * zc + p,),
            device_id_type=pl.DeviceIdType.MESH,
        )
        rdma.start()
        s2.append(rdma)

    red = cb_ref[pl.ds(z * CHUNK, CHUNK), :].astype(jnp.float32)
    for zoff in range(1, N_Z):
        s2[zoff - 1].wait_recv()
        red = red + buf2[zoff - 1].astype(jnp.float32)
    red_ref[...] = red.astype(jnp.bfloat16)
    all4_ref[pl.ds(z * CHUNK, CHUNK), :] = red_ref[...]

    s3 = []
    for zoff in range(1, N_Z):
        zc = lax.rem(z + zoff, N_Z)
        rdma = pltpu.make_async_remote_copy(
            src_ref=red_ref,
            dst_ref=all4_ref.at[pl.ds(z * CHUNK, CHUNK), :],
            send_sem=s3_send.at[zoff - 1],
            recv_sem=s3_recv.at[zoff - 1],
            device_id=(4 * zc + p,),
            device_id_type=pl.DeviceIdType.MESH,
        )
        rdma.start()
        s3.append(rdma)

    s4 = []

    def forward_block(k, zs):
        for poff in range(1, N_P):
            pp = lax.rem(p + poff, N_P)
            rdma = pltpu.make_async_remote_copy(
                src_ref=all4_ref.at[pl.ds(zs * CHUNK, CHUNK), :],
                dst_ref=buf3.at[poff - 1, pl.ds(zs * CHUNK, CHUNK), :],
                send_sem=s4_send.at[poff - 1, k],
                recv_sem=s4_recv.at[poff - 1, k],
                device_id=(4 * z + pp,),
                device_id_type=pl.DeviceIdType.MESH,
            )
            rdma.start()
            s4.append(rdma)

    forward_block(0, z)
    out_ref[pl.ds(z * BAND + p * CHUNK, CHUNK), :] = red_ref[...]

    for zoff in range(1, N_Z):
        zs = lax.rem(z + N_Z - zoff, N_Z)
        rdma = pltpu.make_async_remote_copy(
            src_ref=red_ref,
            dst_ref=all4_ref.at[pl.ds(zs * CHUNK, CHUNK), :],
            send_sem=s2_send.at[zoff - 1],
            recv_sem=s3_recv.at[zoff - 1],
            device_id=(4 * zs + p,),
            device_id_type=pl.DeviceIdType.MESH,
        )
        rdma.wait_recv()
        forward_block(zoff, zs)
        out_ref[pl.ds(zs * BAND + p * CHUNK, CHUNK), :] = (
            all4_ref[pl.ds(zs * CHUNK, CHUNK), :])

    for poff in range(1, N_P):
        ps = lax.rem(p + N_P - poff, N_P)
        for k in range(N_Z):
            zs = z if k == 0 else lax.rem(z + N_Z - k, N_Z)
            rdma = pltpu.make_async_remote_copy(
                src_ref=red_ref,
                dst_ref=buf3.at[poff - 1, pl.ds(zs * CHUNK, CHUNK), :],
                send_sem=s2_send.at[poff - 1],
                recv_sem=s4_recv.at[poff - 1, k],
                device_id=(d,),
                device_id_type=pl.DeviceIdType.MESH,
            )
            rdma.wait_recv()
            out_ref[pl.ds(zs * BAND + ps * CHUNK, CHUNK), :] = (
                buf3[poff - 1, pl.ds(zs * CHUNK, CHUNK), :])

    for rdma in s1 + s2 + s3 + s4:
        rdma.wait_send()


def kernel(x, Wq, K_ext, V_ext, Wo):
    out = pl.pallas_call(
        _body,
        out_shape=jax.ShapeDtypeStruct((SQ, SQ), jnp.bfloat16),
        in_specs=[
            pl.BlockSpec(memory_space=pltpu.VMEM),
            pl.BlockSpec(memory_space=pltpu.VMEM),
            pl.BlockSpec(memory_space=pl.ANY),
            pl.BlockSpec(memory_space=pl.ANY),
            pl.BlockSpec(memory_space=pltpu.VMEM),
        ],
        out_specs=pl.BlockSpec(memory_space=pltpu.VMEM),
        scratch_shapes=[
            pltpu.VMEM((H_LOC, SKV, DH), jnp.float32),
            pltpu.VMEM((H_LOC, SKV, DH), jnp.float32),
            pltpu.VMEM((H_LOC, SKV, DH), jnp.bfloat16),
            pltpu.VMEM((H_LOC, SKV, DH), jnp.bfloat16),
            pltpu.VMEM((SQ, SQ), jnp.bfloat16),
            pltpu.VMEM((SQ, SQ), jnp.bfloat16),
            pltpu.VMEM((N_P - 1, N_Z * CHUNK, SQ), jnp.bfloat16),
            pltpu.VMEM((N_Z * CHUNK, SQ), jnp.bfloat16),
            pltpu.VMEM((N_Z - 1, CHUNK, SQ), jnp.bfloat16),
            pltpu.VMEM((CHUNK, SQ), jnp.bfloat16),
            pltpu.VMEM((N_Z * CHUNK, SQ), jnp.bfloat16),
            pltpu.VMEM((N_P - 1, N_Z * CHUNK, SQ), jnp.bfloat16),
            pltpu.SemaphoreType.DMA((H_LOC,)),
            pltpu.SemaphoreType.DMA((H_LOC,)),
            pltpu.SemaphoreType.DMA((N_P - 1, N_BAND)),
            pltpu.SemaphoreType.DMA((N_P - 1, N_BAND)),
            pltpu.SemaphoreType.DMA((N_Z - 1,)),
            pltpu.SemaphoreType.DMA((N_Z - 1,)),
            pltpu.SemaphoreType.DMA((N_Z - 1,)),
            pltpu.SemaphoreType.DMA((N_Z - 1,)),
            pltpu.SemaphoreType.DMA((N_P - 1, N_Z)),
            pltpu.SemaphoreType.DMA((N_P - 1, N_Z)),
        ],
        compiler_params=pltpu.CompilerParams(
            vmem_limit_bytes=110 * 1024 * 1024,
        ),
    )(x, Wq, K_ext, V_ext, Wo)
    return out[None, :, :]
